# baseline (speedup 1.0000x reference)
import functools

import jax
import jax.numpy as jnp
from jax import lax
from jax.experimental import pallas as pl
from jax.experimental.pallas import tpu as pltpu
from jax.experimental.pallas import tpu_sc as plsc

N_ATOMS = 10000
N_BONDS = 160000
MAX_NB = 16
ATOM_FDIM = 128
BOND_FDIM = 144
HIDDEN = 256
DEPTH = 4
N_MOLS = 100
ATOMS_PER_MOL = 100

HP = HIDDEN // 2
NUM_WORKERS = 32
LANES = 16

_mesh = plsc.VectorSubcoreMesh(core_axis_name="c", subcore_axis_name="s")
_sc_params = pltpu.CompilerParams(needs_layout_passes=False)


def _worker_id():
    return lax.axis_index("s") * 2 + lax.axis_index("c")


CHUNK = 128
NCH = N_BONDS // CHUNK
SLOTS = 40
PAIRS = SLOTS // 2


def _tile_range():
    wid = _worker_id()
    count = 39 + jnp.where(wid < 2, 1, 0)
    base = 39 * wid + jnp.minimum(wid, 2)
    start = jnp.minimum(base, NCH - SLOTS)
    return count, base, start, base - start


def _as_bf16(x_i32):
    return plsc.bitcast(x_i32, jnp.bfloat16)


def _as_i32(x_bf16):
    return plsc.bitcast(x_bf16, jnp.int32)


GS_ATOMS = CHUNK // MAX_NB
GS_K = 4


@functools.partial(
    pl.kernel,
    out_type=jax.ShapeDtypeStruct((N_ATOMS, HP), jnp.int32),
    mesh=_mesh,
    scratch_types=[
        pltpu.VMEM((SLOTS * CHUNK,), jnp.int32),
        pltpu.VMEM((GS_K, CHUNK, HP), jnp.int32),
        pltpu.VMEM((GS_K, GS_ATOMS, HP), jnp.int32),
        [pltpu.SemaphoreType.DMA] * GS_K,
        [pltpu.SemaphoreType.DMA] * GS_K,
    ],
    compiler_params=_sc_params,
)
def _sc_gather_sum(a2b_hbm, msg_hbm, out_hbm, idx_v, rows_v, acc_v,
                   gsems, osems):
    count, base, start, lo = _tile_range()
    pltpu.sync_copy(a2b_hbm.at[pl.ds(start * CHUNK, SLOTS * CHUNK)], idx_v)

    def block(q, _):
        lts = []
        descs = []
        for b in range(GS_K):
            t = GS_K * q + b
            lt = jnp.where(t < count, t, 0)
            lts.append(lt)
            descs.append(pltpu.async_copy(
                msg_hbm.at[idx_v.at[pl.ds((lo + lt) * CHUNK, CHUNK)]],
                rows_v.at[b], gsems[b]))
        stores = []
        for b in range(GS_K):
            descs[b].wait()

            def h_step(g, _):
                col = pl.ds(g * LANES, LANES)
                for a in range(GS_ATOMS):
                    vals = [_as_bf16(rows_v[b, a * MAX_NB + k, col])
                            for k in range(MAX_NB)]
                    while len(vals) > 1:
                        vals = [vals[i] + vals[i + 1]
                                for i in range(0, len(vals), 2)]
                    acc_v[b, a, col] = _as_i32(vals[0])
                return 0

            lax.fori_loop(0, HP // LANES, h_step, 0)
            stores.append(pltpu.async_copy(
                acc_v.at[b],
                out_hbm.at[pl.ds((base + lts[b]) * GS_ATOMS, GS_ATOMS)],
                osems[b]))
        for b in range(GS_K):
            stores[b].wait()
        return 0

    lax.fori_loop(0, SLOTS // GS_K, block, 0)


PRE_CHUNK = 64
PRE_NCH = N_BONDS // PRE_CHUNK
PRE_SLOTS = 80
PRE_PAIRS = PRE_SLOTS // 2


def _pre_tile_range():
    wid = _worker_id()
    count = 78 + jnp.where(wid < 4, 1, 0)
    base = 78 * wid + jnp.minimum(wid, 4)
    start = jnp.minimum(base, PRE_NCH - PRE_SLOTS)
    return count, base, start, base - start


@functools.partial(
    pl.kernel,
    out_type=jax.ShapeDtypeStruct((N_BONDS, HP), jnp.int32),
    mesh=_mesh,
    scratch_types=[
        pltpu.VMEM((PRE_SLOTS * PRE_CHUNK,), jnp.int32),
        pltpu.VMEM((PRE_SLOTS * PRE_CHUNK,), jnp.int32),
        pltpu.VMEM((2, PRE_CHUNK, HP), jnp.int32),
        pltpu.VMEM((2, PRE_CHUNK, HP), jnp.int32),
        pltpu.VMEM_SHARED((N_ATOMS, HP), jnp.int32),
        [pltpu.SemaphoreType.DMA] * 2,
        [pltpu.SemaphoreType.DMA] * 2,
        [pltpu.SemaphoreType.DMA] * 2,
    ],
    compiler_params=_sc_params,
)
def _sc_pre(b2a_hbm, b2revb_hbm, amsg_hbm, msg_hbm, out_hbm,
            ia_v, ib_v, arows_v, brows_v, amsg_spm, asems, bsems, osems):
    count, base, start, lo = _pre_tile_range()

    @pl.when(lax.axis_index("s") == 0)
    def _stage():
        pltpu.sync_copy(amsg_hbm, amsg_spm)

    plsc.subcore_barrier()
    npre = PRE_SLOTS * PRE_CHUNK
    pltpu.sync_copy(b2a_hbm.at[pl.ds(start * PRE_CHUNK, npre)], ia_v)
    pltpu.sync_copy(b2revb_hbm.at[pl.ds(start * PRE_CHUNK, npre)], ib_v)

    def pair(q, _):
        lts = []
        adescs = []
        bdescs = []
        for b in range(2):
            t = 2 * q + b
            lt = jnp.where(t < count, t, 0)
            lts.append(lt)
            sl = pl.ds((lo + lt) * PRE_CHUNK, PRE_CHUNK)
            adescs.append(pltpu.async_copy(
                amsg_spm.at[ia_v.at[sl]], arows_v.at[b], asems[b]))
            bdescs.append(pltpu.async_copy(
                msg_hbm.at[ib_v.at[sl]], brows_v.at[b], bsems[b]))
        stores = []
        for b in range(2):
            adescs[b].wait()
            bdescs[b].wait()

            def r_step(r, _):
                for g in range(HP // LANES):
                    col = pl.ds(g * LANES, LANES)
                    arows_v[b, r, col] = _as_i32(
                        _as_bf16(arows_v[b, r, col])
                        - _as_bf16(brows_v[b, r, col]))
                return 0

            lax.fori_loop(0, PRE_CHUNK, r_step, 0)
            stores.append(pltpu.async_copy(
                arows_v.at[b],
                out_hbm.at[pl.ds((base + lts[b]) * PRE_CHUNK, PRE_CHUNK)],
                osems[b]))
        for b in range(2):
            stores[b].wait()
        return 0

    lax.fori_loop(0, PRE_PAIRS, pair, 0)


_ROWS_BLK = 10000


def _pack_halves(y):
    bits = lax.bitcast_convert_type(y.astype(jnp.bfloat16), jnp.uint16)
    lo = bits[:, :HP].astype(jnp.uint32)
    hi = bits[:, HP:].astype(jnp.uint32)
    return lax.bitcast_convert_type((hi << 16) | lo, jnp.int32)


def _unpack_halves(p):
    lo = lax.bitcast_convert_type(p << 16, jnp.float32)
    hi = lax.bitcast_convert_type(p & jnp.int32(-65536), jnp.float32)
    return lo, hi


def _in_proj_body(fb_ref, wi_ref, inp_ref, msg_ref):
    x = jnp.dot(fb_ref[...].astype(jnp.bfloat16),
                wi_ref[...].astype(jnp.bfloat16),
                preferred_element_type=jnp.float32)
    inp_ref[...] = _pack_halves(x)
    msg_ref[...] = _pack_halves(jnp.maximum(x, 0.0))


def _tc_in_proj(f_bonds, W_i):
    return pl.pallas_call(
        _in_proj_body,
        grid=(N_BONDS // _ROWS_BLK,),
        in_specs=[
            pl.BlockSpec((_ROWS_BLK, BOND_FDIM), lambda i: (i, 0)),
            pl.BlockSpec((BOND_FDIM, HIDDEN), lambda i: (0, 0)),
        ],
        out_specs=[
            pl.BlockSpec((_ROWS_BLK, HP), lambda i: (i, 0)),
            pl.BlockSpec((_ROWS_BLK, HP), lambda i: (i, 0)),
        ],
        out_shape=[
            jax.ShapeDtypeStruct((N_BONDS, HP), jnp.int32),
            jax.ShapeDtypeStruct((N_BONDS, HP), jnp.int32),
        ],
    )(f_bonds, W_i)


def _update_body(pre_ref, inp_ref, wh_ref, out_ref):
    plo, phi = _unpack_halves(pre_ref[...])
    x = jnp.dot(plo.astype(jnp.bfloat16), wh_ref[:HP, :],
                preferred_element_type=jnp.float32)
    x = x + jnp.dot(phi.astype(jnp.bfloat16), wh_ref[HP:, :],
                    preferred_element_type=jnp.float32)
    ilo, ihi = _unpack_halves(inp_ref[...])
    ylo = jnp.maximum(x[:, :HP] + ilo, 0.0)
    yhi = jnp.maximum(x[:, HP:] + ihi, 0.0)
    bl = lax.bitcast_convert_type(ylo.astype(jnp.bfloat16),
                                  jnp.uint16).astype(jnp.uint32)
    bh = lax.bitcast_convert_type(yhi.astype(jnp.bfloat16),
                                  jnp.uint16).astype(jnp.uint32)
    out_ref[...] = lax.bitcast_convert_type((bh << 16) | bl, jnp.int32)


def _tc_update(pre, inp, W_h_b16):
    return pl.pallas_call(
        _update_body,
        grid=(N_BONDS // _ROWS_BLK,),
        in_specs=[
            pl.BlockSpec((_ROWS_BLK, HP), lambda i: (i, 0)),
            pl.BlockSpec((_ROWS_BLK, HP), lambda i: (i, 0)),
            pl.BlockSpec((HIDDEN, HIDDEN), lambda i: (0, 0)),
        ],
        out_specs=pl.BlockSpec((_ROWS_BLK, HP), lambda i: (i, 0)),
        out_shape=jax.ShapeDtypeStruct((N_BONDS, HP), jnp.int32),
    )(pre, inp, W_h_b16)


def _out_body(fa_ref, am_ref, wo_ref, bo_ref, out_ref):
    alo, ahi = _unpack_halves(am_ref[...])
    h = jnp.dot(fa_ref[...], wo_ref[:ATOM_FDIM, :],
                preferred_element_type=jnp.float32)
    h = h + jnp.dot(alo, wo_ref[ATOM_FDIM:ATOM_FDIM + HP, :],
                    preferred_element_type=jnp.float32)
    h = h + jnp.dot(ahi, wo_ref[ATOM_FDIM + HP:, :],
                    preferred_element_type=jnp.float32)
    h = jnp.maximum(h + bo_ref[...], 0.0)
    mol = lax.broadcasted_iota(jnp.int32, (N_MOLS, N_ATOMS), 0)
    atom = lax.broadcasted_iota(jnp.int32, (N_MOLS, N_ATOMS), 1)
    pool = jnp.where(atom // ATOMS_PER_MOL == mol,
                     jnp.float32(1.0 / ATOMS_PER_MOL), jnp.float32(0.0))
    out_ref[...] = jnp.dot(pool, h, preferred_element_type=jnp.float32)


def _tc_out(f_atoms, a_message, W_o, b_o):
    return pl.pallas_call(
        _out_body,
        out_shape=jax.ShapeDtypeStruct((N_MOLS, HIDDEN), jnp.float32),
    )(f_atoms, a_message, W_o, b_o.reshape(1, HIDDEN))


def kernel(f_atoms, f_bonds, a2b, b2a, b2revb, W_i, W_h, W_o, b_o):
    a2b_flat = a2b.reshape(-1).astype(jnp.int32)
    b2a = b2a.astype(jnp.int32)
    b2revb = b2revb.astype(jnp.int32)
    W_h_b16 = W_h.astype(jnp.bfloat16)

    inp, message = _tc_in_proj(f_bonds, W_i)
    for _ in range(DEPTH - 1):
        a_message = _sc_gather_sum(a2b_flat, message)
        pre = _sc_pre(b2a, b2revb, a_message, message)
        message = _tc_update(pre, inp, W_h_b16)
    a_message = _sc_gather_sum(a2b_flat, message)
    return _tc_out(f_atoms, a_message, W_o, b_o)


# speedup vs baseline: 1.0017x; 1.0017x over previous
import functools

import jax
import jax.numpy as jnp
from jax import lax
from jax.experimental import pallas as pl
from jax.experimental.pallas import tpu as pltpu
from jax.experimental.pallas import tpu_sc as plsc

N_ATOMS = 10000
N_BONDS = 160000
MAX_NB = 16
ATOM_FDIM = 128
BOND_FDIM = 144
HIDDEN = 256
DEPTH = 4
N_MOLS = 100
ATOMS_PER_MOL = 100

HP = HIDDEN // 2
NUM_WORKERS = 32
LANES = 16

_mesh = plsc.VectorSubcoreMesh(core_axis_name="c", subcore_axis_name="s")
_sc_params = pltpu.CompilerParams(needs_layout_passes=False)


def _worker_id():
    return lax.axis_index("s") * 2 + lax.axis_index("c")


CHUNK = 128
NCH = N_BONDS // CHUNK
SLOTS = 40
PAIRS = SLOTS // 2


def _tile_range():
    wid = _worker_id()
    count = 39 + jnp.where(wid < 2, 1, 0)
    base = 39 * wid + jnp.minimum(wid, 2)
    start = jnp.minimum(base, NCH - SLOTS)
    return count, base, start, base - start


def _as_bf16(x_i32):
    return plsc.bitcast(x_i32, jnp.bfloat16)


def _as_i32(x_bf16):
    return plsc.bitcast(x_bf16, jnp.int32)


GS_ATOMS = CHUNK // MAX_NB
GS_K = 4


@functools.partial(
    pl.kernel,
    out_type=jax.ShapeDtypeStruct((N_ATOMS, HP), jnp.int32),
    mesh=_mesh,
    scratch_types=[
        pltpu.VMEM((SLOTS * CHUNK,), jnp.int32),
        pltpu.VMEM((GS_K, CHUNK, HP), jnp.int32),
        pltpu.VMEM((GS_K, GS_ATOMS, HP), jnp.int32),
        [pltpu.SemaphoreType.DMA] * GS_K,
        [pltpu.SemaphoreType.DMA] * GS_K,
    ],
    compiler_params=_sc_params,
)
def _sc_gather_sum(a2b_hbm, msg_hbm, out_hbm, idx_v, rows_v, acc_v,
                   gsems, osems):
    count, base, start, lo = _tile_range()
    pltpu.sync_copy(a2b_hbm.at[pl.ds(start * CHUNK, SLOTS * CHUNK)], idx_v)

    def block(q, _):
        lts = []
        descs = []
        for b in range(GS_K):
            t = GS_K * q + b
            lt = jnp.where(t < count, t, 0)
            lts.append(lt)
            descs.append(pltpu.async_copy(
                msg_hbm.at[idx_v.at[pl.ds((lo + lt) * CHUNK, CHUNK)]],
                rows_v.at[b], gsems[b]))
        stores = []
        for b in range(GS_K):
            descs[b].wait()

            def h_step(g, _):
                col = pl.ds(g * LANES, LANES)
                for a in range(GS_ATOMS):
                    vals = [_as_bf16(rows_v[b, a * MAX_NB + k, col])
                            for k in range(MAX_NB)]
                    while len(vals) > 1:
                        vals = [vals[i] + vals[i + 1]
                                for i in range(0, len(vals), 2)]
                    acc_v[b, a, col] = _as_i32(vals[0])
                return 0

            lax.fori_loop(0, HP // LANES, h_step, 0)
            stores.append(pltpu.async_copy(
                acc_v.at[b],
                out_hbm.at[pl.ds((base + lts[b]) * GS_ATOMS, GS_ATOMS)],
                osems[b]))
        for b in range(GS_K):
            stores[b].wait()
        return 0

    lax.fori_loop(0, SLOTS // GS_K, block, 0)


PRE_CHUNK = 64
PRE_NCH = N_BONDS // PRE_CHUNK
PRE_SLOTS = 80
PRE_PAIRS = PRE_SLOTS // 2


def _pre_tile_range():
    wid = _worker_id()
    count = 78 + jnp.where(wid < 4, 1, 0)
    base = 78 * wid + jnp.minimum(wid, 4)
    start = jnp.minimum(base, PRE_NCH - PRE_SLOTS)
    return count, base, start, base - start


@functools.partial(
    pl.kernel,
    out_type=jax.ShapeDtypeStruct((N_BONDS, HP), jnp.int32),
    mesh=_mesh,
    scratch_types=[
        pltpu.VMEM((PRE_SLOTS * PRE_CHUNK,), jnp.int32),
        pltpu.VMEM((PRE_SLOTS * PRE_CHUNK,), jnp.int32),
        pltpu.VMEM((2, PRE_CHUNK, HP), jnp.int32),
        pltpu.VMEM((2, PRE_CHUNK, HP), jnp.int32),
        pltpu.VMEM_SHARED((N_ATOMS, HP), jnp.int32),
        [pltpu.SemaphoreType.DMA] * 2,
        [pltpu.SemaphoreType.DMA] * 2,
        [pltpu.SemaphoreType.DMA] * 2,
    ],
    compiler_params=_sc_params,
)
def _sc_pre(b2a_hbm, b2revb_hbm, amsg_hbm, msg_hbm, out_hbm,
            ia_v, ib_v, arows_v, brows_v, amsg_spm, asems, bsems, osems):
    count, base, start, lo = _pre_tile_range()

    @pl.when(lax.axis_index("s") == 0)
    def _stage():
        pltpu.sync_copy(amsg_hbm, amsg_spm)

    plsc.subcore_barrier()
    npre = PRE_SLOTS * PRE_CHUNK
    pltpu.sync_copy(b2a_hbm.at[pl.ds(start * PRE_CHUNK, npre)], ia_v)
    pltpu.sync_copy(b2revb_hbm.at[pl.ds(start * PRE_CHUNK, npre)], ib_v)

    def pair(q, _):
        lts = []
        adescs = []
        bdescs = []
        for b in range(2):
            t = 2 * q + b
            lt = jnp.where(t < count, t, 0)
            lts.append(lt)
            sl = pl.ds((lo + lt) * PRE_CHUNK, PRE_CHUNK)
            adescs.append(pltpu.async_copy(
                amsg_spm.at[ia_v.at[sl]], arows_v.at[b], asems[b]))
            bdescs.append(pltpu.async_copy(
                msg_hbm.at[ib_v.at[sl]], brows_v.at[b], bsems[b]))
        stores = []
        for b in range(2):
            adescs[b].wait()
            bdescs[b].wait()

            def r_step(r, _):
                for g in range(HP // LANES):
                    col = pl.ds(g * LANES, LANES)
                    arows_v[b, r, col] = _as_i32(
                        _as_bf16(arows_v[b, r, col])
                        - _as_bf16(brows_v[b, r, col]))
                return 0

            lax.fori_loop(0, PRE_CHUNK, r_step, 0)
            stores.append(pltpu.async_copy(
                arows_v.at[b],
                out_hbm.at[pl.ds((base + lts[b]) * PRE_CHUNK, PRE_CHUNK)],
                osems[b]))
        for b in range(2):
            stores[b].wait()
        return 0

    lax.fori_loop(0, PRE_PAIRS, pair, 0)


_ROWS_BLK = 8000


def _pack_halves(y):
    bits = lax.bitcast_convert_type(y.astype(jnp.bfloat16), jnp.uint16)
    lo = bits[:, :HP].astype(jnp.uint32)
    hi = bits[:, HP:].astype(jnp.uint32)
    return lax.bitcast_convert_type((hi << 16) | lo, jnp.int32)


def _unpack_halves(p):
    lo = lax.bitcast_convert_type(p << 16, jnp.float32)
    hi = lax.bitcast_convert_type(p & jnp.int32(-65536), jnp.float32)
    return lo, hi


def _in_proj_body(fb_ref, wi_ref, inp_ref, msg_ref):
    x = jnp.dot(fb_ref[...].astype(jnp.bfloat16),
                wi_ref[...].astype(jnp.bfloat16),
                preferred_element_type=jnp.float32)
    inp_ref[...] = _pack_halves(x)
    msg_ref[...] = _pack_halves(jnp.maximum(x, 0.0))


def _tc_in_proj(f_bonds, W_i):
    return pl.pallas_call(
        _in_proj_body,
        grid=(N_BONDS // _ROWS_BLK,),
        in_specs=[
            pl.BlockSpec((_ROWS_BLK, BOND_FDIM), lambda i: (i, 0)),
            pl.BlockSpec((BOND_FDIM, HIDDEN), lambda i: (0, 0)),
        ],
        out_specs=[
            pl.BlockSpec((_ROWS_BLK, HP), lambda i: (i, 0)),
            pl.BlockSpec((_ROWS_BLK, HP), lambda i: (i, 0)),
        ],
        out_shape=[
            jax.ShapeDtypeStruct((N_BONDS, HP), jnp.int32),
            jax.ShapeDtypeStruct((N_BONDS, HP), jnp.int32),
        ],
    )(f_bonds, W_i)


def _update_body(pre_ref, inp_ref, wh_ref, out_ref):
    plo, phi = _unpack_halves(pre_ref[...])
    x = jnp.dot(plo.astype(jnp.bfloat16), wh_ref[:HP, :],
                preferred_element_type=jnp.float32)
    x = x + jnp.dot(phi.astype(jnp.bfloat16), wh_ref[HP:, :],
                    preferred_element_type=jnp.float32)
    ilo, ihi = _unpack_halves(inp_ref[...])
    ylo = jnp.maximum(x[:, :HP] + ilo, 0.0)
    yhi = jnp.maximum(x[:, HP:] + ihi, 0.0)
    bl = lax.bitcast_convert_type(ylo.astype(jnp.bfloat16),
                                  jnp.uint16).astype(jnp.uint32)
    bh = lax.bitcast_convert_type(yhi.astype(jnp.bfloat16),
                                  jnp.uint16).astype(jnp.uint32)
    out_ref[...] = lax.bitcast_convert_type((bh << 16) | bl, jnp.int32)


def _tc_update(pre, inp, W_h_b16):
    return pl.pallas_call(
        _update_body,
        grid=(N_BONDS // _ROWS_BLK,),
        in_specs=[
            pl.BlockSpec((_ROWS_BLK, HP), lambda i: (i, 0)),
            pl.BlockSpec((_ROWS_BLK, HP), lambda i: (i, 0)),
            pl.BlockSpec((HIDDEN, HIDDEN), lambda i: (0, 0)),
        ],
        out_specs=pl.BlockSpec((_ROWS_BLK, HP), lambda i: (i, 0)),
        out_shape=jax.ShapeDtypeStruct((N_BONDS, HP), jnp.int32),
    )(pre, inp, W_h_b16)


def _out_body(fa_ref, am_ref, wo_ref, bo_ref, out_ref):
    alo, ahi = _unpack_halves(am_ref[...])
    h = jnp.dot(fa_ref[...], wo_ref[:ATOM_FDIM, :],
                preferred_element_type=jnp.float32)
    h = h + jnp.dot(alo, wo_ref[ATOM_FDIM:ATOM_FDIM + HP, :],
                    preferred_element_type=jnp.float32)
    h = h + jnp.dot(ahi, wo_ref[ATOM_FDIM + HP:, :],
                    preferred_element_type=jnp.float32)
    h = jnp.maximum(h + bo_ref[...], 0.0)
    mol = lax.broadcasted_iota(jnp.int32, (N_MOLS, N_ATOMS), 0)
    atom = lax.broadcasted_iota(jnp.int32, (N_MOLS, N_ATOMS), 1)
    pool = jnp.where(atom // ATOMS_PER_MOL == mol,
                     jnp.float32(1.0 / ATOMS_PER_MOL), jnp.float32(0.0))
    out_ref[...] = jnp.dot(pool, h, preferred_element_type=jnp.float32)


def _tc_out(f_atoms, a_message, W_o, b_o):
    return pl.pallas_call(
        _out_body,
        out_shape=jax.ShapeDtypeStruct((N_MOLS, HIDDEN), jnp.float32),
    )(f_atoms, a_message, W_o, b_o.reshape(1, HIDDEN))


def kernel(f_atoms, f_bonds, a2b, b2a, b2revb, W_i, W_h, W_o, b_o):
    a2b_flat = a2b.reshape(-1).astype(jnp.int32)
    b2a = b2a.astype(jnp.int32)
    b2revb = b2revb.astype(jnp.int32)
    W_h_b16 = W_h.astype(jnp.bfloat16)

    inp, message = _tc_in_proj(f_bonds, W_i)
    for _ in range(DEPTH - 1):
        a_message = _sc_gather_sum(a2b_flat, message)
        pre = _sc_pre(b2a, b2revb, a_message, message)
        message = _tc_update(pre, inp, W_h_b16)
    a_message = _sc_gather_sum(a2b_flat, message)
    return _tc_out(f_atoms, a_message, W_o, b_o)


# speedup vs baseline: 1.0154x; 1.0137x over previous
import functools

import jax
import jax.numpy as jnp
from jax import lax
from jax.experimental import pallas as pl
from jax.experimental.pallas import tpu as pltpu
from jax.experimental.pallas import tpu_sc as plsc

N_ATOMS = 10000
N_BONDS = 160000
MAX_NB = 16
ATOM_FDIM = 128
BOND_FDIM = 144
HIDDEN = 256
DEPTH = 4
N_MOLS = 100
ATOMS_PER_MOL = 100

HP = HIDDEN // 2
NUM_WORKERS = 32
LANES = 16

_mesh = plsc.VectorSubcoreMesh(core_axis_name="c", subcore_axis_name="s")
_sc_params = pltpu.CompilerParams(needs_layout_passes=False)


def _worker_id():
    return lax.axis_index("s") * 2 + lax.axis_index("c")


CHUNK = 128
NCH = N_BONDS // CHUNK
SLOTS = 40
PAIRS = SLOTS // 2


def _tile_range():
    wid = _worker_id()
    count = 39 + jnp.where(wid < 2, 1, 0)
    base = 39 * wid + jnp.minimum(wid, 2)
    start = jnp.minimum(base, NCH - SLOTS)
    return count, base, start, base - start


def _as_bf16(x_i32):
    return plsc.bitcast(x_i32, jnp.bfloat16)


def _as_i32(x_bf16):
    return plsc.bitcast(x_bf16, jnp.int32)


GS_ATOMS = CHUNK // MAX_NB
GS_K = 5


@functools.partial(
    pl.kernel,
    out_type=jax.ShapeDtypeStruct((N_ATOMS, HP), jnp.int32),
    mesh=_mesh,
    scratch_types=[
        pltpu.VMEM((SLOTS * CHUNK,), jnp.int32),
        pltpu.VMEM((GS_K, CHUNK, HP), jnp.int32),
        pltpu.VMEM((GS_K, GS_ATOMS, HP), jnp.int32),
        [pltpu.SemaphoreType.DMA] * GS_K,
        [pltpu.SemaphoreType.DMA] * GS_K,
    ],
    compiler_params=_sc_params,
)
def _sc_gather_sum(a2b_hbm, msg_hbm, out_hbm, idx_v, rows_v, acc_v,
                   gsems, osems):
    count, base, start, lo = _tile_range()
    pltpu.sync_copy(a2b_hbm.at[pl.ds(start * CHUNK, SLOTS * CHUNK)], idx_v)

    def block(q, _):
        lts = []
        descs = []
        for b in range(GS_K):
            t = GS_K * q + b
            lt = jnp.where(t < count, t, 0)
            lts.append(lt)
            descs.append(pltpu.async_copy(
                msg_hbm.at[idx_v.at[pl.ds((lo + lt) * CHUNK, CHUNK)]],
                rows_v.at[b], gsems[b]))
        stores = []
        for b in range(GS_K):
            descs[b].wait()

            def h_step(g, _):
                col = pl.ds(g * LANES, LANES)
                for a in range(GS_ATOMS):
                    vals = [_as_bf16(rows_v[b, a * MAX_NB + k, col])
                            for k in range(MAX_NB)]
                    while len(vals) > 1:
                        vals = [vals[i] + vals[i + 1]
                                for i in range(0, len(vals), 2)]
                    acc_v[b, a, col] = _as_i32(vals[0])
                return 0

            lax.fori_loop(0, HP // LANES, h_step, 0)
            stores.append(pltpu.async_copy(
                acc_v.at[b],
                out_hbm.at[pl.ds((base + lts[b]) * GS_ATOMS, GS_ATOMS)],
                osems[b]))
        for b in range(GS_K):
            stores[b].wait()
        return 0

    lax.fori_loop(0, SLOTS // GS_K, block, 0)


PRE_CHUNK = 64
PRE_NCH = N_BONDS // PRE_CHUNK
PRE_SLOTS = 80
PRE_PAIRS = PRE_SLOTS // 2


def _pre_tile_range():
    wid = _worker_id()
    count = 78 + jnp.where(wid < 4, 1, 0)
    base = 78 * wid + jnp.minimum(wid, 4)
    start = jnp.minimum(base, PRE_NCH - PRE_SLOTS)
    return count, base, start, base - start


@functools.partial(
    pl.kernel,
    out_type=jax.ShapeDtypeStruct((N_BONDS, HP), jnp.int32),
    mesh=_mesh,
    scratch_types=[
        pltpu.VMEM((PRE_SLOTS * PRE_CHUNK,), jnp.int32),
        pltpu.VMEM((PRE_SLOTS * PRE_CHUNK,), jnp.int32),
        pltpu.VMEM((2, PRE_CHUNK, HP), jnp.int32),
        pltpu.VMEM((2, PRE_CHUNK, HP), jnp.int32),
        pltpu.VMEM_SHARED((N_ATOMS, HP), jnp.int32),
        [pltpu.SemaphoreType.DMA] * 2,
        [pltpu.SemaphoreType.DMA] * 2,
        [pltpu.SemaphoreType.DMA] * 2,
    ],
    compiler_params=_sc_params,
)
def _sc_pre(b2a_hbm, b2revb_hbm, amsg_hbm, msg_hbm, out_hbm,
            ia_v, ib_v, arows_v, brows_v, amsg_spm, asems, bsems, osems):
    count, base, start, lo = _pre_tile_range()

    @pl.when(lax.axis_index("s") == 0)
    def _stage():
        pltpu.sync_copy(amsg_hbm, amsg_spm)

    plsc.subcore_barrier()
    npre = PRE_SLOTS * PRE_CHUNK
    pltpu.sync_copy(b2a_hbm.at[pl.ds(start * PRE_CHUNK, npre)], ia_v)
    pltpu.sync_copy(b2revb_hbm.at[pl.ds(start * PRE_CHUNK, npre)], ib_v)

    def pair(q, _):
        lts = []
        adescs = []
        bdescs = []
        for b in range(2):
            t = 2 * q + b
            lt = jnp.where(t < count, t, 0)
            lts.append(lt)
            sl = pl.ds((lo + lt) * PRE_CHUNK, PRE_CHUNK)
            adescs.append(pltpu.async_copy(
                amsg_spm.at[ia_v.at[sl]], arows_v.at[b], asems[b]))
            bdescs.append(pltpu.async_copy(
                msg_hbm.at[ib_v.at[sl]], brows_v.at[b], bsems[b]))
        stores = []
        for b in range(2):
            adescs[b].wait()
            bdescs[b].wait()

            def r_step(r, _):
                for g in range(HP // LANES):
                    col = pl.ds(g * LANES, LANES)
                    arows_v[b, r, col] = _as_i32(
                        _as_bf16(arows_v[b, r, col])
                        - _as_bf16(brows_v[b, r, col]))
                return 0

            lax.fori_loop(0, PRE_CHUNK, r_step, 0)
            stores.append(pltpu.async_copy(
                arows_v.at[b],
                out_hbm.at[pl.ds((base + lts[b]) * PRE_CHUNK, PRE_CHUNK)],
                osems[b]))
        for b in range(2):
            stores[b].wait()
        return 0

    lax.fori_loop(0, PRE_PAIRS, pair, 0)


_ROWS_BLK = 8000


def _pack_halves(y):
    bits = lax.bitcast_convert_type(y.astype(jnp.bfloat16), jnp.uint16)
    lo = bits[:, :HP].astype(jnp.uint32)
    hi = bits[:, HP:].astype(jnp.uint32)
    return lax.bitcast_convert_type((hi << 16) | lo, jnp.int32)


def _unpack_halves(p):
    lo = lax.bitcast_convert_type(p << 16, jnp.float32)
    hi = lax.bitcast_convert_type(p & jnp.int32(-65536), jnp.float32)
    return lo, hi


def _in_proj_body(fb_ref, wi_ref, inp_ref, msg_ref):
    x = jnp.dot(fb_ref[...].astype(jnp.bfloat16),
                wi_ref[...].astype(jnp.bfloat16),
                preferred_element_type=jnp.float32)
    inp_ref[...] = _pack_halves(x)
    msg_ref[...] = _pack_halves(jnp.maximum(x, 0.0))


def _tc_in_proj(f_bonds, W_i):
    return pl.pallas_call(
        _in_proj_body,
        grid=(N_BONDS // _ROWS_BLK,),
        in_specs=[
            pl.BlockSpec((_ROWS_BLK, BOND_FDIM), lambda i: (i, 0)),
            pl.BlockSpec((BOND_FDIM, HIDDEN), lambda i: (0, 0)),
        ],
        out_specs=[
            pl.BlockSpec((_ROWS_BLK, HP), lambda i: (i, 0)),
            pl.BlockSpec((_ROWS_BLK, HP), lambda i: (i, 0)),
        ],
        out_shape=[
            jax.ShapeDtypeStruct((N_BONDS, HP), jnp.int32),
            jax.ShapeDtypeStruct((N_BONDS, HP), jnp.int32),
        ],
    )(f_bonds, W_i)


def _update_body(pre_ref, inp_ref, wh_ref, out_ref):
    plo, phi = _unpack_halves(pre_ref[...])
    x = jnp.dot(plo.astype(jnp.bfloat16), wh_ref[:HP, :],
                preferred_element_type=jnp.float32)
    x = x + jnp.dot(phi.astype(jnp.bfloat16), wh_ref[HP:, :],
                    preferred_element_type=jnp.float32)
    ilo, ihi = _unpack_halves(inp_ref[...])
    ylo = jnp.maximum(x[:, :HP] + ilo, 0.0)
    yhi = jnp.maximum(x[:, HP:] + ihi, 0.0)
    bl = lax.bitcast_convert_type(ylo.astype(jnp.bfloat16),
                                  jnp.uint16).astype(jnp.uint32)
    bh = lax.bitcast_convert_type(yhi.astype(jnp.bfloat16),
                                  jnp.uint16).astype(jnp.uint32)
    out_ref[...] = lax.bitcast_convert_type((bh << 16) | bl, jnp.int32)


def _tc_update(pre, inp, W_h_b16):
    return pl.pallas_call(
        _update_body,
        grid=(N_BONDS // _ROWS_BLK,),
        in_specs=[
            pl.BlockSpec((_ROWS_BLK, HP), lambda i: (i, 0)),
            pl.BlockSpec((_ROWS_BLK, HP), lambda i: (i, 0)),
            pl.BlockSpec((HIDDEN, HIDDEN), lambda i: (0, 0)),
        ],
        out_specs=pl.BlockSpec((_ROWS_BLK, HP), lambda i: (i, 0)),
        out_shape=jax.ShapeDtypeStruct((N_BONDS, HP), jnp.int32),
    )(pre, inp, W_h_b16)


def _out_body(fa_ref, am_ref, wo_ref, bo_ref, out_ref):
    alo, ahi = _unpack_halves(am_ref[...])
    h = jnp.dot(fa_ref[...], wo_ref[:ATOM_FDIM, :],
                preferred_element_type=jnp.float32)
    h = h + jnp.dot(alo, wo_ref[ATOM_FDIM:ATOM_FDIM + HP, :],
                    preferred_element_type=jnp.float32)
    h = h + jnp.dot(ahi, wo_ref[ATOM_FDIM + HP:, :],
                    preferred_element_type=jnp.float32)
    h = jnp.maximum(h + bo_ref[...], 0.0)
    mol = lax.broadcasted_iota(jnp.int32, (N_MOLS, N_ATOMS), 0)
    atom = lax.broadcasted_iota(jnp.int32, (N_MOLS, N_ATOMS), 1)
    pool = jnp.where(atom // ATOMS_PER_MOL == mol,
                     jnp.float32(1.0 / ATOMS_PER_MOL), jnp.float32(0.0))
    out_ref[...] = jnp.dot(pool, h, preferred_element_type=jnp.float32)


def _tc_out(f_atoms, a_message, W_o, b_o):
    return pl.pallas_call(
        _out_body,
        out_shape=jax.ShapeDtypeStruct((N_MOLS, HIDDEN), jnp.float32),
    )(f_atoms, a_message, W_o, b_o.reshape(1, HIDDEN))


def kernel(f_atoms, f_bonds, a2b, b2a, b2revb, W_i, W_h, W_o, b_o):
    a2b_flat = a2b.reshape(-1).astype(jnp.int32)
    b2a = b2a.astype(jnp.int32)
    b2revb = b2revb.astype(jnp.int32)
    W_h_b16 = W_h.astype(jnp.bfloat16)

    inp, message = _tc_in_proj(f_bonds, W_i)
    for _ in range(DEPTH - 1):
        a_message = _sc_gather_sum(a2b_flat, message)
        pre = _sc_pre(b2a, b2revb, a_message, message)
        message = _tc_update(pre, inp, W_h_b16)
    a_message = _sc_gather_sum(a2b_flat, message)
    return _tc_out(f_atoms, a_message, W_o, b_o)
